# baseline (speedup 1.0000x reference)
import functools

import jax
import jax.numpy as jnp
from jax import lax
from jax.experimental import pallas as pl
from jax.experimental.pallas import tpu as pltpu
from jax.experimental.pallas import tpu_sc as plsc

_B = 8
_C_OUT = 64
_STRIDE = 12
_N = _B * _C_OUT
_NC = 2
_NS = 16
_NW = _NC * _NS
_PER_W = _N // _NW

_mesh = plsc.VectorSubcoreMesh(core_axis_name="c", subcore_axis_name="s")


@functools.partial(
    pl.kernel,
    out_type=jax.ShapeDtypeStruct((_B, _C_OUT, 224, 224), jnp.float32),
    mesh=_mesh,
    scratch_types=[pltpu.VMEM((224, 224), jnp.float32)],
)
def _sc_copy(x_hbm, o_hbm, buf):
    wid = lax.axis_index("s") * _NC + lax.axis_index("c")
    for k in range(_PER_W):
        p = wid * _PER_W + k
        b = p // _C_OUT
        c = p % _C_OUT
        pltpu.sync_copy(x_hbm.at[b, c * _STRIDE], buf)
        pltpu.sync_copy(buf, o_hbm.at[b, c])


def kernel(x):
    return _sc_copy(x)


# speedup vs baseline: 1.0069x; 1.0069x over previous
import functools

import jax
import jax.numpy as jnp
from jax import lax
from jax.experimental import pallas as pl
from jax.experimental.pallas import tpu as pltpu
from jax.experimental.pallas import tpu_sc as plsc

_B = 8
_C_OUT = 64
_STRIDE = 12
_N = _B * _C_OUT
_NC = 2
_NS = 16
_NW = _NC * _NS
_PER_W = _N // _NW

_mesh = plsc.VectorSubcoreMesh(core_axis_name="c", subcore_axis_name="s")


@functools.partial(
    pl.kernel,
    out_type=jax.ShapeDtypeStruct((_B, _C_OUT, 224, 224), jnp.float32),
    mesh=_mesh,
    scratch_types=[
        pltpu.VMEM((2, 224, 224), jnp.float32),
        pltpu.SemaphoreType.DMA((2,)),
        pltpu.SemaphoreType.DMA((2,)),
    ],
)
def _sc_copy(x_hbm, o_hbm, buf, isems, osems):
    wid = lax.axis_index("s") * _NC + lax.axis_index("c")

    def src(k):
        p = wid * _PER_W + k
        return x_hbm.at[p // _C_OUT, (p % _C_OUT) * _STRIDE]

    def dst(k):
        p = wid * _PER_W + k
        return o_hbm.at[p // _C_OUT, p % _C_OUT]

    cur_in = pltpu.async_copy(src(0), buf.at[0], isems.at[0])
    prev_out = None
    for k in range(_PER_W):
        s = k % 2
        cur_in.wait()
        if prev_out is not None:
            prev_out.wait()
        if k + 1 < _PER_W:
            cur_in = pltpu.async_copy(src(k + 1), buf.at[1 - s], isems.at[1 - s])
        prev_out = pltpu.async_copy(buf.at[s], dst(k), osems.at[s])
    prev_out.wait()


def kernel(x):
    return _sc_copy(x)


# speedup vs baseline: 1.3152x; 1.3062x over previous
import jax
import jax.numpy as jnp
from jax.experimental import pallas as pl
from jax.experimental.pallas import tpu as pltpu

_R = 8
_T = 32
_L = 4


def _copy_kernel(x_ref, o_ref, buf, rsems, wsems):
    reads = [
        pltpu.make_async_copy(
            x_ref.at[i // 4, i % 4, :, 0], buf.at[i % _R], rsems.at[i % _R]
        )
        for i in range(_T)
    ]
    writes = [
        pltpu.make_async_copy(
            buf.at[i % _R], o_ref.at[i // 4, i % 4], wsems.at[i % _R]
        )
        for i in range(_T)
    ]
    for i in range(_T + _L):
        if i < _T:
            if i >= _R:
                writes[i - _R].wait()
            reads[i].start()
        j = i - _L
        if 0 <= j < _T:
            reads[j].wait()
            writes[j].start()
    for i in range(_T - _R, _T):
        writes[i].wait()


def kernel(x):
    xv = x.reshape(8, 4, 16, 12, 224, 224)
    out = pl.pallas_call(
        _copy_kernel,
        in_specs=[pl.BlockSpec(memory_space=pl.ANY)],
        out_specs=pl.BlockSpec(memory_space=pl.ANY),
        out_shape=jax.ShapeDtypeStruct((8, 4, 16, 224, 224), jnp.float32),
        scratch_shapes=[
            pltpu.VMEM((_R, 16, 224, 224), jnp.float32),
            pltpu.SemaphoreType.DMA((_R,)),
            pltpu.SemaphoreType.DMA((_R,)),
        ],
    )(xv)
    return out.reshape(8, 64, 224, 224)
